# baseline (speedup 1.0000x reference)
import functools

import jax
import jax.numpy as jnp
from jax import lax
from jax.experimental import pallas as pl
from jax.experimental.pallas import tpu as pltpu
from jax.experimental.pallas import tpu_sc as plsc

N = 10000
NP = 10240
E = 160000
F = 256
HH = 128
NSC = 2
NT = 16
RPT = NP // NT

EPT_A = E // (NSC * NT)
CH_A = 40
NCH_A = EPT_A // CH_A

W_A = 8

def _deg_body(src_hbm, deg_hbm, idx_v, ones_v, zb_v, acc_sh, sem):
    c = lax.axis_index("c")
    s = lax.axis_index("s")
    one16 = jnp.ones((16,), jnp.float32)
    zero16 = jnp.zeros((16,), jnp.float32)
    ones_v[pl.ds(0, 16)] = one16
    ones_v[pl.ds(16, 16)] = one16
    ones_v[pl.ds(24, 16)] = one16
    for j in range(RPT // 16):
        zb_v[pl.ds(j * 16, 16)] = zero16
    pltpu.sync_copy(src_hbm.at[c, s], idx_v)
    pltpu.sync_copy(zb_v, acc_sh.at[pl.ds(s * RPT, RPT)])
    plsc.subcore_barrier()

    for j in range(W_A):
        pltpu.async_copy(ones_v, acc_sh.at[idx_v.at[j]], sem, add=True)

    def chunk(k, carry):
        pltpu.make_async_copy(ones_v, acc_sh.at[idx_v.at[k]], sem).wait()
        @pl.when(k + W_A < NCH_A)
        def _issue():
            pltpu.async_copy(ones_v, acc_sh.at[idx_v.at[k + W_A]], sem,
                             add=True)
        return carry

    lax.fori_loop(0, NCH_A, chunk, 0)
    plsc.subcore_barrier()
    pltpu.sync_copy(acc_sh.at[pl.ds(s * RPT, RPT)],
                    deg_hbm.at[c, pl.ds(s * RPT, RPT)])


_mesh = plsc.VectorSubcoreMesh(core_axis_name="c", subcore_axis_name="s")

_deg_call = pl.kernel(
    _deg_body,
    out_type=jax.ShapeDtypeStruct((NSC, NP), jnp.float32),
    mesh=_mesh,
    scratch_types=[
        pltpu.VMEM((NCH_A, CH_A), jnp.int32),
        pltpu.VMEM((CH_A,), jnp.float32),
        pltpu.VMEM((RPT,), jnp.float32),
        pltpu.VMEM_SHARED((NP,), jnp.float32),
        pltpu.SemaphoreType.DMA,
    ],
)

CH_M = 128
NCH_M = 80
EPT_M = NCH_M * CH_M
EPAD = NT * EPT_M

def _msg_body(yp_hbm, src_hbm, dst_hbm, out_hbm,
              sidx_v, didx_v, rows_v, zb_v, acc_sh, gsem, isem0, isem1):
    c = lax.axis_index("c")
    s = lax.axis_index("s")
    zero16 = jnp.zeros((16,), jnp.float32)
    for r in range(16):
        for q in range(HH // 16):
            zb_v[r, pl.ds(q * 16, 16)] = zero16
    for j in range(RPT // 16):
        pltpu.sync_copy(zb_v, acc_sh.at[pl.ds(s * RPT + j * 16, 16), :])
    pltpu.sync_copy(src_hbm.at[s, 0], sidx_v.at[0])
    pltpu.sync_copy(dst_hbm.at[s, 0], didx_v.at[0])
    plsc.subcore_barrier()

    pltpu.async_copy(yp_hbm.at[c].at[sidx_v.at[0]], rows_v.at[0], gsem)
    pltpu.async_copy(src_hbm.at[s, 1], sidx_v.at[1], isem1)
    pltpu.async_copy(dst_hbm.at[s, 1], didx_v.at[1], isem1)

    def half_step(k, par, isem_cur, isem_nxt):
        pltpu.make_async_copy(yp_hbm.at[c].at[sidx_v.at[par]],
                              rows_v.at[par], gsem).wait()
        @pl.when(k + 1 < NCH_M)
        def _issue_gather():
            pltpu.make_async_copy(src_hbm.at[s, k + 1],
                                  sidx_v.at[1 - par], isem_nxt).wait()
            pltpu.make_async_copy(dst_hbm.at[s, k + 1],
                                  didx_v.at[1 - par], isem_nxt).wait()
            pltpu.async_copy(yp_hbm.at[c].at[sidx_v.at[1 - par]],
                             rows_v.at[1 - par], gsem)
        pltpu.sync_copy(rows_v.at[par], acc_sh.at[didx_v.at[par]], add=True)
        @pl.when(k + 2 < NCH_M)
        def _issue_idx():
            pltpu.async_copy(src_hbm.at[s, k + 2], sidx_v.at[par], isem_cur)
            pltpu.async_copy(dst_hbm.at[s, k + 2], didx_v.at[par], isem_cur)

    def chunk2(m, carry):
        half_step(2 * m, 0, isem0, isem1)
        half_step(2 * m + 1, 1, isem1, isem0)
        return carry

    lax.fori_loop(0, NCH_M // 2, chunk2, 0)
    plsc.subcore_barrier()
    pltpu.sync_copy(acc_sh.at[pl.ds(s * RPT, RPT), :],
                    out_hbm.at[c, pl.ds(s * RPT, RPT), :])


_msg_call = pl.kernel(
    _msg_body,
    out_type=jax.ShapeDtypeStruct((NSC, NP, HH), jnp.float32),
    mesh=_mesh,
    scratch_types=[
        pltpu.VMEM((2, CH_M), jnp.int32),
        pltpu.VMEM((2, CH_M), jnp.int32),
        pltpu.VMEM((2, CH_M, HH), jnp.float32),
        pltpu.VMEM((16, HH), jnp.float32),
        pltpu.VMEM_SHARED((NP, HH), jnp.float32),
        pltpu.SemaphoreType.DMA,
        pltpu.SemaphoreType.DMA,
        pltpu.SemaphoreType.DMA,
    ],
)

RB = 640
GRID = NP // RB

def _tc_b_body(deg_ref, x_ref, w0_ref, w1_ref, b_ref,
               xw0_ref, yp_ref, dis_ref):
    deg = deg_ref[0] + deg_ref[1]
    dis = jnp.where(deg > 0, lax.rsqrt(jnp.maximum(deg, 1.0)), 0.0)
    dis_ref[...] = dis
    xb = x_ref[...]
    xw0_ref[...] = (jnp.dot(xb, w0_ref[...], preferred_element_type=jnp.float32)
                    + b_ref[...])
    y = dis * jnp.dot(xb, w1_ref[...], preferred_element_type=jnp.float32)
    yp_ref[0] = y[:, :HH]
    yp_ref[1] = y[:, HH:]


def _tc_d_body(xw0_ref, t1_ref, dis_ref, w0_ref, w1_ref, b_ref,
               hw0_ref, y2p_ref):
    dis = dis_ref[...]
    hl = jnp.maximum(xw0_ref[:, :HH] - dis * t1_ref[0], 0.0)
    hr = jnp.maximum(xw0_ref[:, HH:] - dis * t1_ref[1], 0.0)
    hw0 = (jnp.dot(hl, w0_ref[:HH, :], preferred_element_type=jnp.float32)
           + jnp.dot(hr, w0_ref[HH:, :], preferred_element_type=jnp.float32)
           + b_ref[...])
    hw0_ref[...] = hw0
    y2 = dis * (jnp.dot(hl, w1_ref[:HH, :], preferred_element_type=jnp.float32)
                + jnp.dot(hr, w1_ref[HH:, :], preferred_element_type=jnp.float32))
    y2p_ref[0] = y2[:, :HH]
    y2p_ref[1] = y2[:, HH:]


def _tc_f_body(hw0_ref, t2_ref, dis_ref, out_ref):
    dis = dis_ref[...]
    out_ref[:, :HH] = jnp.maximum(hw0_ref[:, :HH] - dis * t2_ref[0], 0.0)
    out_ref[:, HH:] = jnp.maximum(hw0_ref[:, HH:] - dis * t2_ref[1], 0.0)


_spec_rows = pl.BlockSpec((RB, F), lambda i: (i, 0))
_spec_half = pl.BlockSpec((NSC, RB, HH), lambda i: (0, i, 0))
_spec_col = pl.BlockSpec((RB, 1), lambda i: (i, 0))
_spec_w = pl.BlockSpec((F, F), lambda i: (0, 0))
_spec_b = pl.BlockSpec((1, F), lambda i: (0, 0))

_tc_b_call = pl.pallas_call(
    _tc_b_body,
    grid=(GRID,),
    in_specs=[pl.BlockSpec((NSC, RB, 1), lambda i: (0, i, 0)),
              _spec_rows, _spec_w, _spec_w, _spec_b],
    out_specs=[_spec_rows, _spec_half, _spec_col],
    out_shape=[jax.ShapeDtypeStruct((NP, F), jnp.float32),
               jax.ShapeDtypeStruct((NSC, NP, HH), jnp.float32),
               jax.ShapeDtypeStruct((NP, 1), jnp.float32)],
)

_tc_d_call = pl.pallas_call(
    _tc_d_body,
    grid=(GRID,),
    in_specs=[_spec_rows, _spec_half, _spec_col, _spec_w, _spec_w, _spec_b],
    out_specs=[_spec_rows, _spec_half],
    out_shape=[jax.ShapeDtypeStruct((NP, F), jnp.float32),
               jax.ShapeDtypeStruct((NSC, NP, HH), jnp.float32)],
)

_tc_f_call = pl.pallas_call(
    _tc_f_body,
    grid=(GRID,),
    in_specs=[_spec_rows, _spec_half, _spec_col],
    out_specs=_spec_rows,
    out_shape=jax.ShapeDtypeStruct((NP, F), jnp.float32),
)


def kernel(x, edge_index, W0a, W1a, b1, W0b, W1b, b2):
    src = edge_index[0]
    dst = edge_index[1]
    src_a = src.reshape(NSC, NT, NCH_A, CH_A)
    src_m = jnp.pad(src, (0, EPAD - E)).reshape(NT, NCH_M, CH_M)
    dst_m = jnp.pad(dst, (0, EPAD - E),
                    constant_values=N).reshape(NT, NCH_M, CH_M)
    xp = jnp.pad(x, ((0, NP - N), (0, 0)))
    deg2 = _deg_call(src_a)
    xw0, yp, dis = _tc_b_call(deg2[:, :, None], xp, W0a, W1a, b1[None, :])
    t1 = _msg_call(yp, src_m, dst_m)
    hw0, y2p = _tc_d_call(xw0, t1, dis, W0b, W1b, b2[None, :])
    t2 = _msg_call(y2p, src_m, dst_m)
    out = _tc_f_call(hw0, t2, dis)
    return out[:N]


# speedup vs baseline: 1.7009x; 1.7009x over previous
import functools

import jax
import jax.numpy as jnp
from jax import lax
from jax.experimental import pallas as pl
from jax.experimental.pallas import tpu as pltpu
from jax.experimental.pallas import tpu_sc as plsc

N = 10000
NP = 10240
E = 160000
F = 256
HH = 128
NSC = 2
NT = 16
RPT = NP // NT

EPT_A = E // (NSC * NT)
CH_A = 40
NCH_A = EPT_A // CH_A

W_A = 8

def _deg_body(src_hbm, deg_hbm, idx_v, ones_v, zb_v, acc_sh, sem):
    c = lax.axis_index("c")
    s = lax.axis_index("s")
    one16 = jnp.ones((16,), jnp.float32)
    zero16 = jnp.zeros((16,), jnp.float32)
    ones_v[pl.ds(0, 16)] = one16
    ones_v[pl.ds(16, 16)] = one16
    ones_v[pl.ds(24, 16)] = one16
    for j in range(RPT // 16):
        zb_v[pl.ds(j * 16, 16)] = zero16
    pltpu.sync_copy(src_hbm.at[c, s], idx_v)
    pltpu.sync_copy(zb_v, acc_sh.at[pl.ds(s * RPT, RPT)])
    plsc.subcore_barrier()

    for j in range(W_A):
        pltpu.async_copy(ones_v, acc_sh.at[idx_v.at[j]], sem, add=True)

    def chunk(k, carry):
        pltpu.make_async_copy(ones_v, acc_sh.at[idx_v.at[k]], sem).wait()
        @pl.when(k + W_A < NCH_A)
        def _issue():
            pltpu.async_copy(ones_v, acc_sh.at[idx_v.at[k + W_A]], sem,
                             add=True)
        return carry

    lax.fori_loop(0, NCH_A, chunk, 0)
    plsc.subcore_barrier()
    pltpu.sync_copy(acc_sh.at[pl.ds(s * RPT, RPT)],
                    deg_hbm.at[c, pl.ds(s * RPT, RPT)])


_mesh = plsc.VectorSubcoreMesh(core_axis_name="c", subcore_axis_name="s")

_deg_call = pl.kernel(
    _deg_body,
    out_type=jax.ShapeDtypeStruct((NSC, NP), jnp.float32),
    mesh=_mesh,
    scratch_types=[
        pltpu.VMEM((NCH_A, CH_A), jnp.int32),
        pltpu.VMEM((CH_A,), jnp.float32),
        pltpu.VMEM((RPT,), jnp.float32),
        pltpu.VMEM_SHARED((NP,), jnp.float32),
        pltpu.SemaphoreType.DMA,
    ],
)

CH_M = 80
NCH_M = 125
EPT_M = NCH_M * CH_M

def _msg_body(yp_hbm, src_hbm, dst_hbm, out_hbm,
              sidx_v, didx_v, rows_v, zb_v, acc_sh, gsem, dsem):
    c = lax.axis_index("c")
    s = lax.axis_index("s")
    zero16 = jnp.zeros((16,), jnp.float32)
    for r in range(16):
        for q in range(HH // 16):
            zb_v[r, pl.ds(q * 16, 16)] = zero16
    pltpu.sync_copy(src_hbm.at[s], sidx_v)
    for j in range(RPT // 16):
        pltpu.sync_copy(zb_v, acc_sh.at[pl.ds(s * RPT + j * 16, 16), :])
    plsc.subcore_barrier()

    pltpu.async_copy(yp_hbm.at[c].at[sidx_v.at[0]], rows_v.at[0], gsem)
    pltpu.async_copy(dst_hbm.at[s, 0], didx_v.at[0], dsem)

    def chunk(k, carry):
        par = lax.rem(k, 2)
        pltpu.make_async_copy(yp_hbm.at[c].at[sidx_v.at[k]],
                              rows_v.at[par], gsem).wait()
        @pl.when(k + 1 < NCH_M)
        def _issue():
            pltpu.async_copy(yp_hbm.at[c].at[sidx_v.at[k + 1]],
                             rows_v.at[1 - par], gsem)
            pltpu.async_copy(dst_hbm.at[s, k + 1], didx_v.at[1 - par], dsem)
        pltpu.make_async_copy(dst_hbm.at[s, k], didx_v.at[par], dsem).wait()
        pltpu.sync_copy(rows_v.at[par], acc_sh.at[didx_v.at[par]], add=True)
        return carry

    lax.fori_loop(0, NCH_M, chunk, 0)
    plsc.subcore_barrier()
    pltpu.sync_copy(acc_sh.at[pl.ds(s * RPT, RPT), :],
                    out_hbm.at[c, pl.ds(s * RPT, RPT), :])


_msg_call = pl.kernel(
    _msg_body,
    out_type=jax.ShapeDtypeStruct((NSC, NP, HH), jnp.float32),
    mesh=_mesh,
    scratch_types=[
        pltpu.VMEM((NCH_M, CH_M), jnp.int32),
        pltpu.VMEM((2, CH_M), jnp.int32),
        pltpu.VMEM((2, CH_M, HH), jnp.float32),
        pltpu.VMEM((16, HH), jnp.float32),
        pltpu.VMEM_SHARED((NP, HH), jnp.float32),
        pltpu.SemaphoreType.DMA,
        pltpu.SemaphoreType.DMA,
    ],
)

RB = 400
GRID = N // RB

def _tc_b_body(deg_ref, x_ref, w0_ref, w1_ref, b_ref,
               xw0_ref, yp_ref, dis_ref):
    deg = deg_ref[0] + deg_ref[1]
    dis = jnp.where(deg > 0, lax.rsqrt(jnp.maximum(deg, 1.0)), 0.0)
    dis_ref[...] = dis
    xb = x_ref[...]
    xw0_ref[...] = (jnp.dot(xb, w0_ref[...], preferred_element_type=jnp.float32)
                    + b_ref[...])
    y = dis * jnp.dot(xb, w1_ref[...], preferred_element_type=jnp.float32)
    yp_ref[0] = y[:, :HH]
    yp_ref[1] = y[:, HH:]


def _tc_d_body(xw0_ref, t1_ref, dis_ref, w0_ref, w1_ref, b_ref,
               hw0_ref, y2p_ref):
    dis = dis_ref[...]
    hl = jnp.maximum(xw0_ref[:, :HH] - dis * t1_ref[0], 0.0)
    hr = jnp.maximum(xw0_ref[:, HH:] - dis * t1_ref[1], 0.0)
    hw0 = (jnp.dot(hl, w0_ref[:HH, :], preferred_element_type=jnp.float32)
           + jnp.dot(hr, w0_ref[HH:, :], preferred_element_type=jnp.float32)
           + b_ref[...])
    hw0_ref[...] = hw0
    y2 = dis * (jnp.dot(hl, w1_ref[:HH, :], preferred_element_type=jnp.float32)
                + jnp.dot(hr, w1_ref[HH:, :], preferred_element_type=jnp.float32))
    y2p_ref[0] = y2[:, :HH]
    y2p_ref[1] = y2[:, HH:]


def _tc_f_body(hw0_ref, t2_ref, dis_ref, out_ref):
    dis = dis_ref[...]
    out_ref[:, :HH] = jnp.maximum(hw0_ref[:, :HH] - dis * t2_ref[0], 0.0)
    out_ref[:, HH:] = jnp.maximum(hw0_ref[:, HH:] - dis * t2_ref[1], 0.0)


_spec_rows = pl.BlockSpec((RB, F), lambda i: (i, 0))
_spec_half = pl.BlockSpec((NSC, RB, HH), lambda i: (0, i, 0))
_spec_col = pl.BlockSpec((RB, 1), lambda i: (i, 0))
_spec_w = pl.BlockSpec((F, F), lambda i: (0, 0))
_spec_b = pl.BlockSpec((1, F), lambda i: (0, 0))

_tc_b_call = pl.pallas_call(
    _tc_b_body,
    grid=(GRID,),
    in_specs=[pl.BlockSpec((NSC, RB, 1), lambda i: (0, i, 0)),
              _spec_rows, _spec_w, _spec_w, _spec_b],
    out_specs=[_spec_rows, _spec_half, _spec_col],
    out_shape=[jax.ShapeDtypeStruct((N, F), jnp.float32),
               jax.ShapeDtypeStruct((NSC, N, HH), jnp.float32),
               jax.ShapeDtypeStruct((N, 1), jnp.float32)],
)

_tc_d_call = pl.pallas_call(
    _tc_d_body,
    grid=(GRID,),
    in_specs=[_spec_rows, _spec_half, _spec_col, _spec_w, _spec_w, _spec_b],
    out_specs=[_spec_rows, _spec_half],
    out_shape=[jax.ShapeDtypeStruct((N, F), jnp.float32),
               jax.ShapeDtypeStruct((NSC, N, HH), jnp.float32)],
)

_tc_f_call = pl.pallas_call(
    _tc_f_body,
    grid=(GRID,),
    in_specs=[_spec_rows, _spec_half, _spec_col],
    out_specs=_spec_rows,
    out_shape=jax.ShapeDtypeStruct((N, F), jnp.float32),
)


def kernel(x, edge_index, W0a, W1a, b1, W0b, W1b, b2):
    src = edge_index[0]
    dst = edge_index[1]
    src_a = src.reshape(NSC, NT, NCH_A, CH_A)
    src_m = src.reshape(NT, NCH_M, CH_M)
    dst_m = dst.reshape(NT, NCH_M, CH_M)
    deg2 = _deg_call(src_a)
    xw0, yp, dis = _tc_b_call(deg2[:, :, None], x, W0a, W1a, b1[None, :])
    t1 = _msg_call(yp, src_m, dst_m)
    hw0, y2p = _tc_d_call(xw0, t1, dis, W0b, W1b, b2[None, :])
    t2 = _msg_call(y2p, src_m, dst_m)
    return _tc_f_call(hw0, t2, dis)


# speedup vs baseline: 1.7013x; 1.0002x over previous
import functools

import jax
import jax.numpy as jnp
from jax import lax
from jax.experimental import pallas as pl
from jax.experimental.pallas import tpu as pltpu
from jax.experimental.pallas import tpu_sc as plsc

N = 10000
NP = 10240
E = 160000
F = 256
HH = 128
NSC = 2
NT = 16
RPT = NP // NT

EPT_A = E // (NSC * NT)
CH_A = 40
NCH_A = EPT_A // CH_A

W_A = 8

def _deg_body(src_hbm, deg_hbm, idx_v, ones_v, zb_v, acc_sh, sem):
    c = lax.axis_index("c")
    s = lax.axis_index("s")
    one16 = jnp.ones((16,), jnp.float32)
    zero16 = jnp.zeros((16,), jnp.float32)
    ones_v[pl.ds(0, 16)] = one16
    ones_v[pl.ds(16, 16)] = one16
    ones_v[pl.ds(24, 16)] = one16
    for j in range(RPT // 16):
        zb_v[pl.ds(j * 16, 16)] = zero16
    pltpu.sync_copy(src_hbm.at[c, s], idx_v)
    pltpu.sync_copy(zb_v, acc_sh.at[pl.ds(s * RPT, RPT)])
    plsc.subcore_barrier()

    for j in range(W_A):
        pltpu.async_copy(ones_v, acc_sh.at[idx_v.at[j]], sem, add=True)

    def chunk(k, carry):
        pltpu.make_async_copy(ones_v, acc_sh.at[idx_v.at[k]], sem).wait()
        @pl.when(k + W_A < NCH_A)
        def _issue():
            pltpu.async_copy(ones_v, acc_sh.at[idx_v.at[k + W_A]], sem,
                             add=True)
        return carry

    lax.fori_loop(0, NCH_A, chunk, 0)
    plsc.subcore_barrier()
    pltpu.sync_copy(acc_sh.at[pl.ds(s * RPT, RPT)],
                    deg_hbm.at[c, pl.ds(s * RPT, RPT)])


_mesh = plsc.VectorSubcoreMesh(core_axis_name="c", subcore_axis_name="s")

_deg_call = pl.kernel(
    _deg_body,
    out_type=jax.ShapeDtypeStruct((NSC, NP), jnp.float32),
    mesh=_mesh,
    scratch_types=[
        pltpu.VMEM((NCH_A, CH_A), jnp.int32),
        pltpu.VMEM((CH_A,), jnp.float32),
        pltpu.VMEM((RPT,), jnp.float32),
        pltpu.VMEM_SHARED((NP,), jnp.float32),
        pltpu.SemaphoreType.DMA,
    ],
)

CH_M = 80
NCH_M = 125
EPT_M = NCH_M * CH_M

def _msg_body(yp_hbm, src_hbm, dst_hbm, out_hbm,
              sidx_v, didx_v, rows_v, zb_v, acc_sh, gsem, dsem, ssem):
    c = lax.axis_index("c")
    s = lax.axis_index("s")
    zero16 = jnp.zeros((16,), jnp.float32)
    for r in range(16):
        for q in range(HH // 16):
            zb_v[r, pl.ds(q * 16, 16)] = zero16
    pltpu.sync_copy(src_hbm.at[s], sidx_v)
    for j in range(RPT // 16):
        pltpu.sync_copy(zb_v, acc_sh.at[pl.ds(s * RPT + j * 16, 16), :])
    plsc.subcore_barrier()

    pltpu.async_copy(yp_hbm.at[c].at[sidx_v.at[0]], rows_v.at[0], gsem)
    pltpu.async_copy(dst_hbm.at[s, 0], didx_v.at[0], dsem)

    def chunk(k, carry):
        par = lax.rem(k, 2)
        pltpu.make_async_copy(yp_hbm.at[c].at[sidx_v.at[k]],
                              rows_v.at[par], gsem).wait()
        @pl.when(k >= 1)
        def _wait_sc():
            pltpu.make_async_copy(rows_v.at[1 - par],
                                  acc_sh.at[didx_v.at[1 - par]], ssem).wait()
        @pl.when(k + 1 < NCH_M)
        def _issue():
            pltpu.async_copy(yp_hbm.at[c].at[sidx_v.at[k + 1]],
                             rows_v.at[1 - par], gsem)
            pltpu.async_copy(dst_hbm.at[s, k + 1], didx_v.at[1 - par], dsem)
        pltpu.make_async_copy(dst_hbm.at[s, k], didx_v.at[par], dsem).wait()
        pltpu.async_copy(rows_v.at[par], acc_sh.at[didx_v.at[par]], ssem,
                         add=True)
        return carry

    lax.fori_loop(0, NCH_M, chunk, 0)
    pltpu.make_async_copy(rows_v.at[0], acc_sh.at[didx_v.at[0]], ssem).wait()
    plsc.subcore_barrier()
    pltpu.sync_copy(acc_sh.at[pl.ds(s * RPT, RPT), :],
                    out_hbm.at[c, pl.ds(s * RPT, RPT), :])


_msg_call = pl.kernel(
    _msg_body,
    out_type=jax.ShapeDtypeStruct((NSC, NP, HH), jnp.float32),
    mesh=_mesh,
    scratch_types=[
        pltpu.VMEM((NCH_M, CH_M), jnp.int32),
        pltpu.VMEM((2, CH_M), jnp.int32),
        pltpu.VMEM((2, CH_M, HH), jnp.float32),
        pltpu.VMEM((16, HH), jnp.float32),
        pltpu.VMEM_SHARED((NP, HH), jnp.float32),
        pltpu.SemaphoreType.DMA,
        pltpu.SemaphoreType.DMA,
        pltpu.SemaphoreType.DMA,
    ],
)

RB = 400
GRID = N // RB

def _tc_b_body(deg_ref, x_ref, w0_ref, w1_ref, b_ref,
               xw0_ref, yp_ref, dis_ref):
    deg = deg_ref[0] + deg_ref[1]
    dis = jnp.where(deg > 0, lax.rsqrt(jnp.maximum(deg, 1.0)), 0.0)
    dis_ref[...] = dis
    xb = x_ref[...]
    xw0_ref[...] = (jnp.dot(xb, w0_ref[...], preferred_element_type=jnp.float32)
                    + b_ref[...])
    y = dis * jnp.dot(xb, w1_ref[...], preferred_element_type=jnp.float32)
    yp_ref[0] = y[:, :HH]
    yp_ref[1] = y[:, HH:]


def _tc_d_body(xw0_ref, t1_ref, dis_ref, w0_ref, w1_ref, b_ref,
               hw0_ref, y2p_ref):
    dis = dis_ref[...]
    hl = jnp.maximum(xw0_ref[:, :HH] - dis * t1_ref[0], 0.0)
    hr = jnp.maximum(xw0_ref[:, HH:] - dis * t1_ref[1], 0.0)
    hw0 = (jnp.dot(hl, w0_ref[:HH, :], preferred_element_type=jnp.float32)
           + jnp.dot(hr, w0_ref[HH:, :], preferred_element_type=jnp.float32)
           + b_ref[...])
    hw0_ref[...] = hw0
    y2 = dis * (jnp.dot(hl, w1_ref[:HH, :], preferred_element_type=jnp.float32)
                + jnp.dot(hr, w1_ref[HH:, :], preferred_element_type=jnp.float32))
    y2p_ref[0] = y2[:, :HH]
    y2p_ref[1] = y2[:, HH:]


def _tc_f_body(hw0_ref, t2_ref, dis_ref, out_ref):
    dis = dis_ref[...]
    out_ref[:, :HH] = jnp.maximum(hw0_ref[:, :HH] - dis * t2_ref[0], 0.0)
    out_ref[:, HH:] = jnp.maximum(hw0_ref[:, HH:] - dis * t2_ref[1], 0.0)


_spec_rows = pl.BlockSpec((RB, F), lambda i: (i, 0))
_spec_half = pl.BlockSpec((NSC, RB, HH), lambda i: (0, i, 0))
_spec_col = pl.BlockSpec((RB, 1), lambda i: (i, 0))
_spec_w = pl.BlockSpec((F, F), lambda i: (0, 0))
_spec_b = pl.BlockSpec((1, F), lambda i: (0, 0))

_tc_b_call = pl.pallas_call(
    _tc_b_body,
    grid=(GRID,),
    in_specs=[pl.BlockSpec((NSC, RB, 1), lambda i: (0, i, 0)),
              _spec_rows, _spec_w, _spec_w, _spec_b],
    out_specs=[_spec_rows, _spec_half, _spec_col],
    out_shape=[jax.ShapeDtypeStruct((N, F), jnp.float32),
               jax.ShapeDtypeStruct((NSC, N, HH), jnp.float32),
               jax.ShapeDtypeStruct((N, 1), jnp.float32)],
)

_tc_d_call = pl.pallas_call(
    _tc_d_body,
    grid=(GRID,),
    in_specs=[_spec_rows, _spec_half, _spec_col, _spec_w, _spec_w, _spec_b],
    out_specs=[_spec_rows, _spec_half],
    out_shape=[jax.ShapeDtypeStruct((N, F), jnp.float32),
               jax.ShapeDtypeStruct((NSC, N, HH), jnp.float32)],
)

_tc_f_call = pl.pallas_call(
    _tc_f_body,
    grid=(GRID,),
    in_specs=[_spec_rows, _spec_half, _spec_col],
    out_specs=_spec_rows,
    out_shape=jax.ShapeDtypeStruct((N, F), jnp.float32),
)


def kernel(x, edge_index, W0a, W1a, b1, W0b, W1b, b2):
    src = edge_index[0]
    dst = edge_index[1]
    src_a = src.reshape(NSC, NT, NCH_A, CH_A)
    src_m = src.reshape(NT, NCH_M, CH_M)
    dst_m = dst.reshape(NT, NCH_M, CH_M)
    deg2 = _deg_call(src_a)
    xw0, yp, dis = _tc_b_call(deg2[:, :, None], x, W0a, W1a, b1[None, :])
    t1 = _msg_call(yp, src_m, dst_m)
    hw0, y2p = _tc_d_call(xw0, t1, dis, W0b, W1b, b2[None, :])
    t2 = _msg_call(y2p, src_m, dst_m)
    return _tc_f_call(hw0, t2, dis)


# speedup vs baseline: 1.8820x; 1.1062x over previous
import functools

import jax
import jax.numpy as jnp
from jax import lax
from jax.experimental import pallas as pl
from jax.experimental.pallas import tpu as pltpu
from jax.experimental.pallas import tpu_sc as plsc

N = 10000
NP = 10240
E = 160000
F = 256
HH = 128
NSC = 2
NT = 16
RPT = NP // NT

EPT_A = E // (NSC * NT)
CH_A = 40
NCH_A = EPT_A // CH_A

W_A = 8

def _deg_body(src_hbm, deg_hbm, idx_v, ones_v, zb_v, acc_sh, sem):
    c = lax.axis_index("c")
    s = lax.axis_index("s")
    one16 = jnp.ones((16,), jnp.float32)
    zero16 = jnp.zeros((16,), jnp.float32)
    ones_v[pl.ds(0, 16)] = one16
    ones_v[pl.ds(16, 16)] = one16
    ones_v[pl.ds(24, 16)] = one16
    for j in range(RPT // 16):
        zb_v[pl.ds(j * 16, 16)] = zero16
    pltpu.sync_copy(src_hbm.at[c, s], idx_v)
    pltpu.sync_copy(zb_v, acc_sh.at[pl.ds(s * RPT, RPT)])
    plsc.subcore_barrier()

    for j in range(W_A):
        pltpu.async_copy(ones_v, acc_sh.at[idx_v.at[j]], sem, add=True)

    def chunk(k, carry):
        pltpu.make_async_copy(ones_v, acc_sh.at[idx_v.at[k]], sem).wait()
        @pl.when(k + W_A < NCH_A)
        def _issue():
            pltpu.async_copy(ones_v, acc_sh.at[idx_v.at[k + W_A]], sem,
                             add=True)
        return carry

    lax.fori_loop(0, NCH_A, chunk, 0)
    plsc.subcore_barrier()
    pltpu.sync_copy(acc_sh.at[pl.ds(s * RPT, RPT)],
                    deg_hbm.at[c, pl.ds(s * RPT, RPT)])


_mesh = plsc.VectorSubcoreMesh(core_axis_name="c", subcore_axis_name="s")

_deg_call = pl.kernel(
    _deg_body,
    out_type=jax.ShapeDtypeStruct((NSC, NP), jnp.float32),
    mesh=_mesh,
    scratch_types=[
        pltpu.VMEM((NCH_A, CH_A), jnp.int32),
        pltpu.VMEM((CH_A,), jnp.float32),
        pltpu.VMEM((RPT,), jnp.float32),
        pltpu.VMEM_SHARED((NP,), jnp.float32),
        pltpu.SemaphoreType.DMA,
    ],
)

CH_M = 40
NCH_M = 250
EPT_M = NCH_M * CH_M
NBUF = 3
ZMAIN = 632
ZTAIL = N - (NT - 1) * ZMAIN

def _msg_body(yp_hbm, src_hbm, dst_hbm, out_hbm,
              sidx_v, didx_v, rows_v, zb_v, acc_sh, gsem, dsem, ssem):
    c = lax.axis_index("c")
    s = lax.axis_index("s")
    zero16 = jnp.zeros((16,), jnp.float32)
    for r in range(8):
        for q in range(HH // 16):
            zb_v[r, pl.ds(q * 16, 16)] = zero16
    pltpu.sync_copy(src_hbm.at[s], sidx_v)
    @pl.when(s < NT - 1)
    def _zero_main():
        for j in range(ZMAIN // 8):
            pltpu.sync_copy(zb_v, acc_sh.at[pl.ds(s * ZMAIN + j * 8, 8), :])
    @pl.when(s == NT - 1)
    def _zero_tail():
        for j in range(ZTAIL // 8):
            pltpu.sync_copy(zb_v,
                            acc_sh.at[pl.ds((NT - 1) * ZMAIN + j * 8, 8), :])
    plsc.subcore_barrier()

    for j in range(NBUF - 1):
        pltpu.async_copy(yp_hbm.at[c].at[sidx_v.at[j]], rows_v.at[j], gsem)
        pltpu.async_copy(dst_hbm.at[s, j], didx_v.at[j], dsem)

    def chunk(k, carry):
        b = lax.rem(k, NBUF)
        bp = lax.rem(k + NBUF - 1, NBUF)
        pltpu.make_async_copy(yp_hbm.at[c].at[sidx_v.at[k]],
                              rows_v.at[b], gsem).wait()
        pltpu.make_async_copy(dst_hbm.at[s, k], didx_v.at[b], dsem).wait()
        @pl.when(k >= 1)
        def _wait_sc():
            pltpu.make_async_copy(rows_v.at[bp],
                                  acc_sh.at[didx_v.at[bp]], ssem).wait()
        pltpu.async_copy(rows_v.at[b], acc_sh.at[didx_v.at[b]], ssem,
                         add=True)
        @pl.when(k + NBUF - 1 < NCH_M)
        def _issue():
            pltpu.async_copy(yp_hbm.at[c].at[sidx_v.at[k + NBUF - 1]],
                             rows_v.at[bp], gsem)
            pltpu.async_copy(dst_hbm.at[s, k + NBUF - 1], didx_v.at[bp],
                             dsem)
        return carry

    lax.fori_loop(0, NCH_M, chunk, 0)
    _last = (NCH_M - 1) % NBUF
    pltpu.make_async_copy(rows_v.at[_last], acc_sh.at[didx_v.at[_last]],
                          ssem).wait()
    plsc.subcore_barrier()
    @pl.when(s < NT - 1)
    def _out_main():
        pltpu.sync_copy(acc_sh.at[pl.ds(s * ZMAIN, ZMAIN), :],
                        out_hbm.at[c, pl.ds(s * ZMAIN, ZMAIN), :])
    @pl.when(s == NT - 1)
    def _out_tail():
        pltpu.sync_copy(acc_sh.at[pl.ds((NT - 1) * ZMAIN, ZTAIL), :],
                        out_hbm.at[c, pl.ds((NT - 1) * ZMAIN, ZTAIL), :])


_msg_call = pl.kernel(
    _msg_body,
    out_type=jax.ShapeDtypeStruct((NSC, N, HH), jnp.float32),
    mesh=_mesh,
    scratch_types=[
        pltpu.VMEM((NCH_M, CH_M), jnp.int32),
        pltpu.VMEM((NBUF, CH_M), jnp.int32),
        pltpu.VMEM((NBUF, CH_M, HH), jnp.float32),
        pltpu.VMEM((8, HH), jnp.float32),
        pltpu.VMEM_SHARED((N, HH), jnp.float32),
        pltpu.SemaphoreType.DMA,
        pltpu.SemaphoreType.DMA,
        pltpu.SemaphoreType.DMA,
    ],
)

RB = 400
GRID = N // RB

def _tc_b_body(deg_ref, x_ref, w0_ref, w1_ref, b_ref,
               xw0_ref, yp_ref, dis_ref):
    deg = deg_ref[0] + deg_ref[1]
    dis = jnp.where(deg > 0, lax.rsqrt(jnp.maximum(deg, 1.0)), 0.0)
    dis_ref[...] = dis
    xb = x_ref[...]
    xw0_ref[...] = (jnp.dot(xb, w0_ref[...], preferred_element_type=jnp.float32)
                    + b_ref[...])
    y = dis * jnp.dot(xb, w1_ref[...], preferred_element_type=jnp.float32)
    yp_ref[0] = y[:, :HH]
    yp_ref[1] = y[:, HH:]


def _tc_d_body(xw0_ref, t1_ref, dis_ref, w0_ref, w1_ref, b_ref,
               hw0_ref, y2p_ref):
    dis = dis_ref[...]
    hl = jnp.maximum(xw0_ref[:, :HH] - dis * t1_ref[0], 0.0)
    hr = jnp.maximum(xw0_ref[:, HH:] - dis * t1_ref[1], 0.0)
    hw0 = (jnp.dot(hl, w0_ref[:HH, :], preferred_element_type=jnp.float32)
           + jnp.dot(hr, w0_ref[HH:, :], preferred_element_type=jnp.float32)
           + b_ref[...])
    hw0_ref[...] = hw0
    y2 = dis * (jnp.dot(hl, w1_ref[:HH, :], preferred_element_type=jnp.float32)
                + jnp.dot(hr, w1_ref[HH:, :], preferred_element_type=jnp.float32))
    y2p_ref[0] = y2[:, :HH]
    y2p_ref[1] = y2[:, HH:]


def _tc_f_body(hw0_ref, t2_ref, dis_ref, out_ref):
    dis = dis_ref[...]
    out_ref[:, :HH] = jnp.maximum(hw0_ref[:, :HH] - dis * t2_ref[0], 0.0)
    out_ref[:, HH:] = jnp.maximum(hw0_ref[:, HH:] - dis * t2_ref[1], 0.0)


_spec_rows = pl.BlockSpec((RB, F), lambda i: (i, 0))
_spec_half = pl.BlockSpec((NSC, RB, HH), lambda i: (0, i, 0))
_spec_col = pl.BlockSpec((RB, 1), lambda i: (i, 0))
_spec_w = pl.BlockSpec((F, F), lambda i: (0, 0))
_spec_b = pl.BlockSpec((1, F), lambda i: (0, 0))

_tc_b_call = pl.pallas_call(
    _tc_b_body,
    grid=(GRID,),
    in_specs=[pl.BlockSpec((NSC, RB, 1), lambda i: (0, i, 0)),
              _spec_rows, _spec_w, _spec_w, _spec_b],
    out_specs=[_spec_rows, _spec_half, _spec_col],
    out_shape=[jax.ShapeDtypeStruct((N, F), jnp.float32),
               jax.ShapeDtypeStruct((NSC, N, HH), jnp.float32),
               jax.ShapeDtypeStruct((N, 1), jnp.float32)],
)

_tc_d_call = pl.pallas_call(
    _tc_d_body,
    grid=(GRID,),
    in_specs=[_spec_rows, _spec_half, _spec_col, _spec_w, _spec_w, _spec_b],
    out_specs=[_spec_rows, _spec_half],
    out_shape=[jax.ShapeDtypeStruct((N, F), jnp.float32),
               jax.ShapeDtypeStruct((NSC, N, HH), jnp.float32)],
)

_tc_f_call = pl.pallas_call(
    _tc_f_body,
    grid=(GRID,),
    in_specs=[_spec_rows, _spec_half, _spec_col],
    out_specs=_spec_rows,
    out_shape=jax.ShapeDtypeStruct((N, F), jnp.float32),
)


def kernel(x, edge_index, W0a, W1a, b1, W0b, W1b, b2):
    src = edge_index[0]
    dst = edge_index[1]
    src_a = src.reshape(NSC, NT, NCH_A, CH_A)
    src_m = src.reshape(NT, NCH_M, CH_M)
    dst_m = dst.reshape(NT, NCH_M, CH_M)
    deg2 = _deg_call(src_a)
    xw0, yp, dis = _tc_b_call(deg2[:, :, None], x, W0a, W1a, b1[None, :])
    t1 = _msg_call(yp, src_m, dst_m)
    hw0, y2p = _tc_d_call(xw0, t1, dis, W0b, W1b, b2[None, :])
    t2 = _msg_call(y2p, src_m, dst_m)
    return _tc_f_call(hw0, t2, dis)
